# baseline (device time: 118664 ns/iter reference)
import jax
import jax.numpy as jnp
from jax import lax
from jax.experimental import pallas as pl
from jax.experimental.pallas import tpu as pltpu

N_DEV = 4
B = 2
SQ = 512
DM = 768
HQ = 32
DH = 64
HLOC = 8
CLOC = HLOC * DH
SKV = 512
WIN = 128
KVN = SQ + WIN
MESH = pl.DeviceIdType.MESH


def kernel(x, Wq, K_ext, V_ext, Wo):
    K2 = K_ext.reshape(B, SKV, HQ * DH)
    V2 = V_ext.reshape(B, SKV, HQ * DH)

    def body(x_ref, wq_ref, k_ref, v_ref, wo_ref, out_ref,
             k_stage, v_stage, k_buf, v_buf, q_buf, ctx_buf, bias,
             acc, kv_recv, kv_send, ar_recv, ar_send, dummy):
        my = lax.axis_index("i")
        bf = jnp.bfloat16

        bsem = pltpu.get_barrier_semaphore()
        for off in (1, 2, 3):
            pl.semaphore_signal(bsem, inc=1, device_id=((my + off) % N_DEV,),
                                device_id_type=MESH)
        pl.semaphore_wait(bsem, N_DEV - 1)

        @pl.when(my == 0)
        def _():
            k_stage[...] = k_ref[...].astype(bf)
            v_stage[...] = v_ref[...].astype(bf)
            k_buf[:, 0:SKV, :] = k_stage[:, :, 0:CLOC]
            v_buf[:, 0:SKV, :] = v_stage[:, :, 0:CLOC]
            for j, d in enumerate((1, 2, 3)):
                pltpu.make_async_remote_copy(
                    src_ref=k_stage.at[:, :, d * CLOC:(d + 1) * CLOC],
                    dst_ref=k_buf.at[:, 0:SKV, :],
                    send_sem=kv_send.at[j], recv_sem=kv_recv.at[0],
                    device_id=(d,), device_id_type=MESH,
                ).start()
                pltpu.make_async_remote_copy(
                    src_ref=v_stage.at[:, :, d * CLOC:(d + 1) * CLOC],
                    dst_ref=v_buf.at[:, 0:SKV, :],
                    send_sem=kv_send.at[3 + j], recv_sem=kv_recv.at[1],
                    device_id=(d,), device_id_type=MESH,
                ).start()

        @pl.when(my == 1)
        def _():
            k_stage[:, 0:WIN, :] = k_ref[:, 0:WIN, :].astype(bf)
            v_stage[:, 0:WIN, :] = v_ref[:, 0:WIN, :].astype(bf)
            k_buf[:, SKV:KVN, :] = k_stage[:, 0:WIN, CLOC:2 * CLOC]
            v_buf[:, SKV:KVN, :] = v_stage[:, 0:WIN, CLOC:2 * CLOC]
            for j, d in enumerate((0, 2, 3)):
                pltpu.make_async_remote_copy(
                    src_ref=k_stage.at[:, 0:WIN, d * CLOC:(d + 1) * CLOC],
                    dst_ref=k_buf.at[:, SKV:KVN, :],
                    send_sem=kv_send.at[j], recv_sem=kv_recv.at[2],
                    device_id=(d,), device_id_type=MESH,
                ).start()
                pltpu.make_async_remote_copy(
                    src_ref=v_stage.at[:, 0:WIN, d * CLOC:(d + 1) * CLOC],
                    dst_ref=v_buf.at[:, SKV:KVN, :],
                    send_sem=kv_send.at[3 + j], recv_sem=kv_recv.at[3],
                    device_id=(d,), device_id_type=MESH,
                ).start()

        wqb = wq_ref[...].astype(bf)
        for b in range(B):
            q_buf[b] = jnp.dot(x_ref[b].astype(bf), wqb,
                               preferred_element_type=jnp.float32).astype(bf)
        qi = lax.broadcasted_iota(jnp.int32, (SQ, KVN), 0)
        ki = lax.broadcasted_iota(jnp.int32, (SQ, KVN), 1)
        bias[...] = jnp.where(jnp.abs(qi - ki) <= WIN, 0.0, -1e9).astype(
            jnp.float32)

        def wait_recv(src, dst, sem):
            pltpu.make_async_remote_copy(
                src_ref=src, dst_ref=dst, send_sem=dummy.at[0], recv_sem=sem,
                device_id=(0,), device_id_type=MESH,
            ).wait_recv()

        @pl.when(my != 0)
        def _():
            wait_recv(k_stage.at[:, :, 0:CLOC], k_buf.at[:, 0:SKV, :],
                      kv_recv.at[0])
            wait_recv(v_stage.at[:, :, 0:CLOC], v_buf.at[:, 0:SKV, :],
                      kv_recv.at[1])

        @pl.when(my != 1)
        def _():
            wait_recv(k_stage.at[:, 0:WIN, 0:CLOC], k_buf.at[:, SKV:KVN, :],
                      kv_recv.at[2])
            wait_recv(v_stage.at[:, 0:WIN, 0:CLOC], v_buf.at[:, SKV:KVN, :],
                      kv_recv.at[3])

        for b in range(B):
            for h in range(HLOC):
                c0, c1 = h * DH, (h + 1) * DH
                s = lax.dot_general(
                    q_buf[b, :, c0:c1], k_buf[b, :, c0:c1],
                    (((1,), (1,)), ((), ())),
                    preferred_element_type=jnp.float32)
                s = s * 0.125 + bias[...]
                m = jnp.max(s, axis=1, keepdims=True)
                e = jnp.exp(s - m)
                w = (e / jnp.sum(e, axis=1, keepdims=True)).astype(bf)
                ctx = jnp.dot(w, v_buf[b, :, c0:c1],
                              preferred_element_type=jnp.float32)
                ctx_buf[b, :, c0:c1] = ctx.astype(bf)

        wob = wo_ref[...].astype(bf)
        for b in range(B):
            acc[0, b] = jnp.dot(ctx_buf[b], wob,
                                preferred_element_type=jnp.float32).astype(bf)

        for off in (1, 2, 3):
            pltpu.make_async_remote_copy(
                src_ref=acc.at[0], dst_ref=acc.at[N_DEV - off],
                send_sem=ar_send.at[off - 1], recv_sem=ar_recv.at[N_DEV - off],
                device_id=((my + off) % N_DEV,), device_id_type=MESH,
            ).start()
        for r in (1, 2, 3):
            pltpu.make_async_remote_copy(
                src_ref=acc.at[0], dst_ref=acc.at[r],
                send_sem=dummy.at[0], recv_sem=ar_recv.at[r],
                device_id=(0,), device_id_type=MESH,
            ).wait_recv()

        for b in range(B):
            out_ref[b] = (acc[0, b].astype(jnp.float32)
                          + acc[1, b].astype(jnp.float32)
                          + acc[2, b].astype(jnp.float32)
                          + acc[3, b].astype(jnp.float32))

        @pl.when(my == 0)
        def _():
            for j, d in enumerate((1, 2, 3)):
                pltpu.make_async_remote_copy(
                    src_ref=k_stage.at[:, :, d * CLOC:(d + 1) * CLOC],
                    dst_ref=k_buf.at[:, 0:SKV, :],
                    send_sem=kv_send.at[j], recv_sem=kv_recv.at[0],
                    device_id=(d,), device_id_type=MESH,
                ).wait_send()
                pltpu.make_async_remote_copy(
                    src_ref=v_stage.at[:, :, d * CLOC:(d + 1) * CLOC],
                    dst_ref=v_buf.at[:, 0:SKV, :],
                    send_sem=kv_send.at[3 + j], recv_sem=kv_recv.at[1],
                    device_id=(d,), device_id_type=MESH,
                ).wait_send()

        @pl.when(my == 1)
        def _():
            for j, d in enumerate((0, 2, 3)):
                pltpu.make_async_remote_copy(
                    src_ref=k_stage.at[:, 0:WIN, d * CLOC:(d + 1) * CLOC],
                    dst_ref=k_buf.at[:, SKV:KVN, :],
                    send_sem=kv_send.at[j], recv_sem=kv_recv.at[2],
                    device_id=(d,), device_id_type=MESH,
                ).wait_send()
                pltpu.make_async_remote_copy(
                    src_ref=v_stage.at[:, 0:WIN, d * CLOC:(d + 1) * CLOC],
                    dst_ref=v_buf.at[:, SKV:KVN, :],
                    send_sem=kv_send.at[3 + j], recv_sem=kv_recv.at[3],
                    device_id=(d,), device_id_type=MESH,
                ).wait_send()

        for off in (1, 2, 3):
            pltpu.make_async_remote_copy(
                src_ref=acc.at[0], dst_ref=acc.at[N_DEV - off],
                send_sem=ar_send.at[off - 1], recv_sem=ar_recv.at[N_DEV - off],
                device_id=((my + off) % N_DEV,), device_id_type=MESH,
            ).wait_send()

    return pl.pallas_call(
        body,
        out_shape=jax.ShapeDtypeStruct((B, SQ, DM), jnp.float32),
        in_specs=[pl.BlockSpec(memory_space=pltpu.VMEM)] * 5,
        out_specs=pl.BlockSpec(memory_space=pltpu.VMEM),
        scratch_shapes=[
            pltpu.VMEM((B, SKV, HQ * DH), jnp.bfloat16),
            pltpu.VMEM((B, SKV, HQ * DH), jnp.bfloat16),
            pltpu.VMEM((B, KVN, CLOC), jnp.bfloat16),
            pltpu.VMEM((B, KVN, CLOC), jnp.bfloat16),
            pltpu.VMEM((B, SQ, CLOC), jnp.bfloat16),
            pltpu.VMEM((B, SQ, CLOC), jnp.bfloat16),
            pltpu.VMEM((SQ, KVN), jnp.float32),
            pltpu.VMEM((N_DEV, B, SQ, DM), jnp.bfloat16),
            pltpu.SemaphoreType.DMA((4,)),
            pltpu.SemaphoreType.DMA((6,)),
            pltpu.SemaphoreType.DMA((4,)),
            pltpu.SemaphoreType.DMA((3,)),
            pltpu.SemaphoreType.DMA((1,)),
        ],
        compiler_params=pltpu.CompilerParams(collective_id=0),
    )(x, Wq, K2, V2, Wo)


# device time: 112010 ns/iter; 1.0594x vs baseline; 1.0594x over previous
import jax
import jax.numpy as jnp
from jax import lax
from jax.experimental import pallas as pl
from jax.experimental.pallas import tpu as pltpu

N_DEV = 4
B = 2
SQ = 512
DM = 768
HQ = 32
DH = 64
HLOC = 8
CLOC = HLOC * DH
SKV = 512
WIN = 128
KVN = SQ + WIN
CH = 256
BANDS = ((0, 384), (128, 640))
MESH = pl.DeviceIdType.MESH


def kernel(x, Wq, K_ext, V_ext, Wo):
    K2 = K_ext.reshape(B, SKV, HQ * DH)
    V2 = V_ext.reshape(B, SKV, HQ * DH)

    def body(x_ref, wq_ref, k_ref, v_ref, wo_ref, out_ref,
             k_stage, v_stage, k_buf, v_buf, q_buf, ctx_buf, bias0, bias1,
             acc, kv_recv, kv_send, ar_recv, ar_send, dummy):
        my = lax.axis_index("i")
        bf = jnp.bfloat16

        bsem = pltpu.get_barrier_semaphore()
        for off in (1, 2, 3):
            pl.semaphore_signal(bsem, inc=1, device_id=((my + off) % N_DEV,),
                                device_id_type=MESH)
        pl.semaphore_wait(bsem, N_DEV - 1)

        @pl.when(my == 0)
        def _():
            k_stage[...] = k_ref[...].astype(bf)
            for j, d in enumerate((1, 2, 3)):
                pltpu.make_async_remote_copy(
                    src_ref=k_stage.at[:, :, d * CLOC:(d + 1) * CLOC],
                    dst_ref=k_buf.at[:, 0:SKV, :],
                    send_sem=kv_send.at[j], recv_sem=kv_recv.at[0],
                    device_id=(d,), device_id_type=MESH,
                ).start()
            v_stage[...] = v_ref[...].astype(bf)
            for j, d in enumerate((1, 2, 3)):
                pltpu.make_async_remote_copy(
                    src_ref=v_stage.at[:, :, d * CLOC:(d + 1) * CLOC],
                    dst_ref=v_buf.at[:, 0:SKV, :],
                    send_sem=kv_send.at[3 + j], recv_sem=kv_recv.at[1],
                    device_id=(d,), device_id_type=MESH,
                ).start()
            k_buf[:, 0:SKV, :] = k_stage[:, :, 0:CLOC]
            v_buf[:, 0:SKV, :] = v_stage[:, :, 0:CLOC]

        @pl.when(my == 1)
        def _():
            k_stage[:, 0:WIN, :] = k_ref[:, 0:WIN, :].astype(bf)
            v_stage[:, 0:WIN, :] = v_ref[:, 0:WIN, :].astype(bf)
            for j, d in enumerate((0, 2, 3)):
                pltpu.make_async_remote_copy(
                    src_ref=k_stage.at[:, 0:WIN, d * CLOC:(d + 1) * CLOC],
                    dst_ref=k_buf.at[:, SKV:KVN, :],
                    send_sem=kv_send.at[j], recv_sem=kv_recv.at[2],
                    device_id=(d,), device_id_type=MESH,
                ).start()
                pltpu.make_async_remote_copy(
                    src_ref=v_stage.at[:, 0:WIN, d * CLOC:(d + 1) * CLOC],
                    dst_ref=v_buf.at[:, SKV:KVN, :],
                    send_sem=kv_send.at[3 + j], recv_sem=kv_recv.at[3],
                    device_id=(d,), device_id_type=MESH,
                ).start()
            k_buf[:, SKV:KVN, :] = k_stage[:, 0:WIN, CLOC:2 * CLOC]
            v_buf[:, SKV:KVN, :] = v_stage[:, 0:WIN, CLOC:2 * CLOC]

        wqb = wq_ref[...].astype(bf)
        for b in range(B):
            q_buf[b] = jnp.dot(x_ref[b].astype(bf), wqb,
                               preferred_element_type=jnp.float32).astype(bf)
        for c, bias in ((0, bias0), (1, bias1)):
            lo, hi = BANDS[c]
            qi = c * CH + lax.broadcasted_iota(jnp.int32, (CH, hi - lo), 0)
            ki = lo + lax.broadcasted_iota(jnp.int32, (CH, hi - lo), 1)
            bias[...] = jnp.where(jnp.abs(qi - ki) <= WIN, 0.0, -1e9).astype(
                jnp.float32)

        def wait_recv(src, dst, sem):
            pltpu.make_async_remote_copy(
                src_ref=src, dst_ref=dst, send_sem=dummy.at[0], recv_sem=sem,
                device_id=(0,), device_id_type=MESH,
            ).wait_recv()

        def attn_chunk(c, bias):
            lo, hi = BANDS[c]
            r0, r1 = c * CH, (c + 1) * CH
            for b in range(B):
                for h in range(HLOC):
                    c0, c1 = h * DH, (h + 1) * DH
                    s = lax.dot_general(
                        q_buf[b, r0:r1, c0:c1], k_buf[b, lo:hi, c0:c1],
                        (((1,), (1,)), ((), ())),
                        preferred_element_type=jnp.float32)
                    s = s * 0.125 + bias[...]
                    m = jnp.max(s, axis=1, keepdims=True)
                    e = jnp.exp(s - m)
                    w = (e / jnp.sum(e, axis=1, keepdims=True)).astype(bf)
                    ctx = jnp.dot(w, v_buf[b, lo:hi, c0:c1],
                                  preferred_element_type=jnp.float32)
                    ctx_buf[b, r0:r1, c0:c1] = ctx.astype(bf)
            wob = wo_ref[...].astype(bf)
            for b in range(B):
                acc[0, b, r0:r1] = jnp.dot(
                    ctx_buf[b, r0:r1, :], wob,
                    preferred_element_type=jnp.float32).astype(bf)
            for off in (1, 2, 3):
                r = N_DEV - off
                pltpu.make_async_remote_copy(
                    src_ref=acc.at[0, :, r0:r1, :],
                    dst_ref=acc.at[r, :, r0:r1, :],
                    send_sem=ar_send.at[3 * c + off - 1],
                    recv_sem=ar_recv.at[4 * c + r],
                    device_id=((my + off) % N_DEV,), device_id_type=MESH,
                ).start()

        def reduce_chunk(c):
            r0, r1 = c * CH, (c + 1) * CH
            for r in (1, 2, 3):
                wait_recv(acc.at[0, :, r0:r1, :], acc.at[r, :, r0:r1, :],
                          ar_recv.at[4 * c + r])
            for b in range(B):
                out_ref[b, r0:r1] = (acc[0, b, r0:r1].astype(jnp.float32)
                                     + acc[1, b, r0:r1].astype(jnp.float32)
                                     + acc[2, b, r0:r1].astype(jnp.float32)
                                     + acc[3, b, r0:r1].astype(jnp.float32))

        @pl.when(my != 0)
        def _():
            wait_recv(k_stage.at[:, :, 0:CLOC], k_buf.at[:, 0:SKV, :],
                      kv_recv.at[0])
            wait_recv(v_stage.at[:, :, 0:CLOC], v_buf.at[:, 0:SKV, :],
                      kv_recv.at[1])

        attn_chunk(0, bias0)

        @pl.when(my != 1)
        def _():
            wait_recv(k_stage.at[:, 0:WIN, 0:CLOC], k_buf.at[:, SKV:KVN, :],
                      kv_recv.at[2])
            wait_recv(v_stage.at[:, 0:WIN, 0:CLOC], v_buf.at[:, SKV:KVN, :],
                      kv_recv.at[3])

        attn_chunk(1, bias1)

        reduce_chunk(0)
        reduce_chunk(1)

        @pl.when(my == 0)
        def _():
            for j, d in enumerate((1, 2, 3)):
                pltpu.make_async_remote_copy(
                    src_ref=k_stage.at[:, :, d * CLOC:(d + 1) * CLOC],
                    dst_ref=k_buf.at[:, 0:SKV, :],
                    send_sem=kv_send.at[j], recv_sem=kv_recv.at[0],
                    device_id=(d,), device_id_type=MESH,
                ).wait_send()
                pltpu.make_async_remote_copy(
                    src_ref=v_stage.at[:, :, d * CLOC:(d + 1) * CLOC],
                    dst_ref=v_buf.at[:, 0:SKV, :],
                    send_sem=kv_send.at[3 + j], recv_sem=kv_recv.at[1],
                    device_id=(d,), device_id_type=MESH,
                ).wait_send()

        @pl.when(my == 1)
        def _():
            for j, d in enumerate((0, 2, 3)):
                pltpu.make_async_remote_copy(
                    src_ref=k_stage.at[:, 0:WIN, d * CLOC:(d + 1) * CLOC],
                    dst_ref=k_buf.at[:, SKV:KVN, :],
                    send_sem=kv_send.at[j], recv_sem=kv_recv.at[2],
                    device_id=(d,), device_id_type=MESH,
                ).wait_send()
                pltpu.make_async_remote_copy(
                    src_ref=v_stage.at[:, 0:WIN, d * CLOC:(d + 1) * CLOC],
                    dst_ref=v_buf.at[:, SKV:KVN, :],
                    send_sem=kv_send.at[3 + j], recv_sem=kv_recv.at[3],
                    device_id=(d,), device_id_type=MESH,
                ).wait_send()

        for c in range(2):
            r0, r1 = c * CH, (c + 1) * CH
            for off in (1, 2, 3):
                r = N_DEV - off
                pltpu.make_async_remote_copy(
                    src_ref=acc.at[0, :, r0:r1, :],
                    dst_ref=acc.at[r, :, r0:r1, :],
                    send_sem=ar_send.at[3 * c + off - 1],
                    recv_sem=ar_recv.at[4 * c + r],
                    device_id=((my + off) % N_DEV,), device_id_type=MESH,
                ).wait_send()

    return pl.pallas_call(
        body,
        out_shape=jax.ShapeDtypeStruct((B, SQ, DM), jnp.float32),
        in_specs=[pl.BlockSpec(memory_space=pltpu.VMEM)] * 5,
        out_specs=pl.BlockSpec(memory_space=pltpu.VMEM),
        scratch_shapes=[
            pltpu.VMEM((B, SKV, HQ * DH), jnp.bfloat16),
            pltpu.VMEM((B, SKV, HQ * DH), jnp.bfloat16),
            pltpu.VMEM((B, KVN, CLOC), jnp.bfloat16),
            pltpu.VMEM((B, KVN, CLOC), jnp.bfloat16),
            pltpu.VMEM((B, SQ, CLOC), jnp.bfloat16),
            pltpu.VMEM((B, SQ, CLOC), jnp.bfloat16),
            pltpu.VMEM((CH, BANDS[0][1] - BANDS[0][0]), jnp.float32),
            pltpu.VMEM((CH, BANDS[1][1] - BANDS[1][0]), jnp.float32),
            pltpu.VMEM((N_DEV, B, SQ, DM), jnp.bfloat16),
            pltpu.SemaphoreType.DMA((4,)),
            pltpu.SemaphoreType.DMA((6,)),
            pltpu.SemaphoreType.DMA((8,)),
            pltpu.SemaphoreType.DMA((6,)),
            pltpu.SemaphoreType.DMA((1,)),
        ],
        compiler_params=pltpu.CompilerParams(collective_id=0),
    )(x, Wq, K2, V2, Wo)


# device time: 99106 ns/iter; 1.1973x vs baseline; 1.1302x over previous
import jax
import jax.numpy as jnp
from jax import lax
from jax.experimental import pallas as pl
from jax.experimental.pallas import tpu as pltpu

N_DEV = 4
B = 2
SQ = 512
DM = 768
HQ = 32
DH = 64
HLOC = 8
CLOC = HLOC * DH
SKV = 512
WIN = 128
KVN = SQ + WIN
CH = 256
BANDS = ((0, 384), (128, 640))
B0 = BANDS[0][1]
MESH = pl.DeviceIdType.MESH


def kernel(x, Wq, K_ext, V_ext, Wo):
    K2 = K_ext.reshape(B, SKV, HQ * DH)
    V2 = V_ext.reshape(B, SKV, HQ * DH)

    def body(x_ref, wq_ref, k_ref, v_ref, wo_ref, out_ref,
             k_stage, v_stage, k_buf, v_buf, q_buf, ctx_buf, bias0, bias1,
             p_send, st1, s2, st2, kv_recv, kv_send, ar_recv, ar_send,
             dummy):
        my = lax.axis_index("i")
        bf = jnp.bfloat16

        bsem = pltpu.get_barrier_semaphore()
        for off in (1, 2, 3):
            pl.semaphore_signal(bsem, inc=1, device_id=((my + off) % N_DEV,),
                                device_id_type=MESH)
        pl.semaphore_wait(bsem, N_DEV - 1)

        @pl.when(my == 0)
        def _():
            k_stage[...] = k_ref[...].astype(bf)
            for j, d in enumerate((1, 2, 3)):
                pltpu.make_async_remote_copy(
                    src_ref=k_stage.at[:, 0:B0, d * CLOC:(d + 1) * CLOC],
                    dst_ref=k_buf.at[:, 0:B0, :],
                    send_sem=kv_send.at[4 * j], recv_sem=kv_recv.at[0],
                    device_id=(d,), device_id_type=MESH,
                ).start()
            v_stage[...] = v_ref[...].astype(bf)
            for j, d in enumerate((1, 2, 3)):
                pltpu.make_async_remote_copy(
                    src_ref=v_stage.at[:, 0:B0, d * CLOC:(d + 1) * CLOC],
                    dst_ref=v_buf.at[:, 0:B0, :],
                    send_sem=kv_send.at[4 * j + 1], recv_sem=kv_recv.at[1],
                    device_id=(d,), device_id_type=MESH,
                ).start()
                pltpu.make_async_remote_copy(
                    src_ref=k_stage.at[:, B0:SKV, d * CLOC:(d + 1) * CLOC],
                    dst_ref=k_buf.at[:, B0:SKV, :],
                    send_sem=kv_send.at[4 * j + 2], recv_sem=kv_recv.at[2],
                    device_id=(d,), device_id_type=MESH,
                ).start()
                pltpu.make_async_remote_copy(
                    src_ref=v_stage.at[:, B0:SKV, d * CLOC:(d + 1) * CLOC],
                    dst_ref=v_buf.at[:, B0:SKV, :],
                    send_sem=kv_send.at[4 * j + 3], recv_sem=kv_recv.at[3],
                    device_id=(d,), device_id_type=MESH,
                ).start()
            k_buf[:, 0:SKV, :] = k_stage[:, :, 0:CLOC]
            v_buf[:, 0:SKV, :] = v_stage[:, :, 0:CLOC]

        @pl.when(my == 1)
        def _():
            k_stage[:, 0:WIN, :] = k_ref[:, 0:WIN, :].astype(bf)
            v_stage[:, 0:WIN, :] = v_ref[:, 0:WIN, :].astype(bf)
            for j, d in enumerate((0, 2, 3)):
                pltpu.make_async_remote_copy(
                    src_ref=k_stage.at[:, 0:WIN, d * CLOC:(d + 1) * CLOC],
                    dst_ref=k_buf.at[:, SKV:KVN, :],
                    send_sem=kv_send.at[4 * j], recv_sem=kv_recv.at[4],
                    device_id=(d,), device_id_type=MESH,
                ).start()
                pltpu.make_async_remote_copy(
                    src_ref=v_stage.at[:, 0:WIN, d * CLOC:(d + 1) * CLOC],
                    dst_ref=v_buf.at[:, SKV:KVN, :],
                    send_sem=kv_send.at[4 * j + 1], recv_sem=kv_recv.at[5],
                    device_id=(d,), device_id_type=MESH,
                ).start()
            k_buf[:, SKV:KVN, :] = k_stage[:, 0:WIN, CLOC:2 * CLOC]
            v_buf[:, SKV:KVN, :] = v_stage[:, 0:WIN, CLOC:2 * CLOC]

        wqb = wq_ref[...].astype(bf)
        for b in range(B):
            q_buf[b] = jnp.dot(x_ref[b].astype(bf), wqb,
                               preferred_element_type=jnp.float32).astype(bf)
        for c, bias in ((0, bias0), (1, bias1)):
            lo, hi = BANDS[c]
            qi = c * CH + lax.broadcasted_iota(jnp.int32, (CH, hi - lo), 0)
            ki = lo + lax.broadcasted_iota(jnp.int32, (CH, hi - lo), 1)
            bias[...] = jnp.where(jnp.abs(qi - ki) <= WIN, 0.0, -1e9).astype(
                jnp.float32)

        def wait_recv(src, dst, sem):
            pltpu.make_async_remote_copy(
                src_ref=src, dst_ref=dst, send_sem=dummy.at[0], recv_sem=sem,
                device_id=(0,), device_id_type=MESH,
            ).wait_recv()

        def attn_chunk(c, bias):
            lo, hi = BANDS[c]
            r0, r1 = c * CH, (c + 1) * CH
            for b in range(B):
                for h in range(HLOC):
                    c0, c1 = h * DH, (h + 1) * DH
                    s = lax.dot_general(
                        q_buf[b, r0:r1, c0:c1], k_buf[b, lo:hi, c0:c1],
                        (((1,), (1,)), ((), ())),
                        preferred_element_type=jnp.float32)
                    s = s * 0.125 + bias[...]
                    m = jnp.max(s, axis=1, keepdims=True)
                    e = jnp.exp(s - m)
                    w = (e / jnp.sum(e, axis=1, keepdims=True)).astype(bf)
                    ctx = jnp.dot(w, v_buf[b, lo:hi, c0:c1],
                                  preferred_element_type=jnp.float32)
                    ctx_buf[b, r0:r1, c0:c1] = ctx.astype(bf)
            wob = wo_ref[...].astype(bf)
            for b in range(B):
                p_send[b, r0:r1] = jnp.dot(
                    ctx_buf[b, r0:r1, :], wob,
                    preferred_element_type=jnp.float32).astype(bf)
            pltpu.make_async_remote_copy(
                src_ref=p_send.at[:, r0:r1, :], dst_ref=st1.at[:, r0:r1, :],
                send_sem=ar_send.at[2 * c], recv_sem=ar_recv.at[2 * c],
                device_id=(my ^ 1,), device_id_type=MESH,
            ).start()

        def ar_stage2(c):
            r0, r1 = c * CH, (c + 1) * CH
            wait_recv(p_send.at[:, r0:r1, :], st1.at[:, r0:r1, :],
                      ar_recv.at[2 * c])
            for b in range(B):
                s2[b, r0:r1] = (p_send[b, r0:r1].astype(jnp.float32)
                                + st1[b, r0:r1].astype(jnp.float32)).astype(bf)
            pltpu.make_async_remote_copy(
                src_ref=s2.at[:, r0:r1, :], dst_ref=st2.at[:, r0:r1, :],
                send_sem=ar_send.at[2 * c + 1], recv_sem=ar_recv.at[2 * c + 1],
                device_id=(3 - my,), device_id_type=MESH,
            ).start()

        def ar_finish(c):
            r0, r1 = c * CH, (c + 1) * CH
            wait_recv(s2.at[:, r0:r1, :], st2.at[:, r0:r1, :],
                      ar_recv.at[2 * c + 1])
            for b in range(B):
                out_ref[b, r0:r1] = (s2[b, r0:r1].astype(jnp.float32)
                                     + st2[b, r0:r1].astype(jnp.float32))

        @pl.when(my != 0)
        def _():
            wait_recv(k_stage.at[:, 0:B0, 0:CLOC], k_buf.at[:, 0:B0, :],
                      kv_recv.at[0])
            wait_recv(v_stage.at[:, 0:B0, 0:CLOC], v_buf.at[:, 0:B0, :],
                      kv_recv.at[1])

        attn_chunk(0, bias0)

        @pl.when(my != 0)
        def _():
            wait_recv(k_stage.at[:, B0:SKV, 0:CLOC], k_buf.at[:, B0:SKV, :],
                      kv_recv.at[2])
            wait_recv(v_stage.at[:, B0:SKV, 0:CLOC], v_buf.at[:, B0:SKV, :],
                      kv_recv.at[3])

        @pl.when(my != 1)
        def _():
            wait_recv(k_stage.at[:, 0:WIN, 0:CLOC], k_buf.at[:, SKV:KVN, :],
                      kv_recv.at[4])
            wait_recv(v_stage.at[:, 0:WIN, 0:CLOC], v_buf.at[:, SKV:KVN, :],
                      kv_recv.at[5])

        attn_chunk(1, bias1)

        ar_stage2(0)
        ar_stage2(1)
        ar_finish(0)
        ar_finish(1)

        @pl.when(my == 0)
        def _():
            for j in range(3):
                for p in range(2):
                    pltpu.make_async_remote_copy(
                        src_ref=k_stage.at[:, 0:B0, 0:CLOC],
                        dst_ref=k_buf.at[:, 0:B0, :],
                        send_sem=kv_send.at[4 * j + p],
                        recv_sem=kv_recv.at[0],
                        device_id=(1,), device_id_type=MESH,
                    ).wait_send()
                    pltpu.make_async_remote_copy(
                        src_ref=k_stage.at[:, B0:SKV, 0:CLOC],
                        dst_ref=k_buf.at[:, B0:SKV, :],
                        send_sem=kv_send.at[4 * j + 2 + p],
                        recv_sem=kv_recv.at[2],
                        device_id=(1,), device_id_type=MESH,
                    ).wait_send()

        @pl.when(my == 1)
        def _():
            for j in range(3):
                for p in range(2):
                    pltpu.make_async_remote_copy(
                        src_ref=k_stage.at[:, 0:WIN, 0:CLOC],
                        dst_ref=k_buf.at[:, SKV:KVN, :],
                        send_sem=kv_send.at[4 * j + p],
                        recv_sem=kv_recv.at[4],
                        device_id=(0,), device_id_type=MESH,
                    ).wait_send()

        for i in range(4):
            pltpu.make_async_remote_copy(
                src_ref=p_send.at[:, 0:CH, :], dst_ref=st1.at[:, 0:CH, :],
                send_sem=ar_send.at[i], recv_sem=ar_recv.at[0],
                device_id=(my ^ 1,), device_id_type=MESH,
            ).wait_send()

    return pl.pallas_call(
        body,
        out_shape=jax.ShapeDtypeStruct((B, SQ, DM), jnp.float32),
        in_specs=[pl.BlockSpec(memory_space=pltpu.VMEM)] * 5,
        out_specs=pl.BlockSpec(memory_space=pltpu.VMEM),
        scratch_shapes=[
            pltpu.VMEM((B, SKV, HQ * DH), jnp.bfloat16),
            pltpu.VMEM((B, SKV, HQ * DH), jnp.bfloat16),
            pltpu.VMEM((B, KVN, CLOC), jnp.bfloat16),
            pltpu.VMEM((B, KVN, CLOC), jnp.bfloat16),
            pltpu.VMEM((B, SQ, CLOC), jnp.bfloat16),
            pltpu.VMEM((B, SQ, CLOC), jnp.bfloat16),
            pltpu.VMEM((CH, BANDS[0][1] - BANDS[0][0]), jnp.float32),
            pltpu.VMEM((CH, BANDS[1][1] - BANDS[1][0]), jnp.float32),
            pltpu.VMEM((B, SQ, DM), jnp.bfloat16),
            pltpu.VMEM((B, SQ, DM), jnp.bfloat16),
            pltpu.VMEM((B, SQ, DM), jnp.bfloat16),
            pltpu.VMEM((B, SQ, DM), jnp.bfloat16),
            pltpu.SemaphoreType.DMA((6,)),
            pltpu.SemaphoreType.DMA((12,)),
            pltpu.SemaphoreType.DMA((4,)),
            pltpu.SemaphoreType.DMA((4,)),
            pltpu.SemaphoreType.DMA((1,)),
        ],
        compiler_params=pltpu.CompilerParams(collective_id=0),
    )(x, Wq, K2, V2, Wo)
